# baseline (speedup 1.0000x reference)
import functools

import jax
import jax.numpy as jnp
from jax import lax
from jax.experimental import pallas as pl
from jax.experimental.pallas import tpu as pltpu
from jax.experimental.pallas import tpu_sc as plsc

_NUM_CORES = 2
_NUM_SUBCORES = 16
_NUM_WORKERS = _NUM_CORES * _NUM_SUBCORES

_BATCH = 16384
_DIM = 64
_CBLK = 128
_ROWS_PER_W = _BATCH // _NUM_WORKERS
_LANES = 16
_RING = 4


def _sgat(ref, k):
    return plsc.load_gather(ref, [jnp.full((_LANES,), k, jnp.int32)])[0]


def _gather_body(
    bl_h, si_h, ks_h, ke_h, bc_h, tt_hbm, outs_hbm,
    bl_v, si_v, ks_v, ke_v, bc_v, ring_v, dst_v, sem0, sem1,
):
    wid = lax.axis_index("s") * _NUM_CORES + lax.axis_index("c")
    pltpu.sync_copy(bl_h.at[wid], bl_v)
    pltpu.sync_copy(si_h.at[wid], si_v)
    pltpu.sync_copy(ks_h.at[wid], ks_v)
    pltpu.sync_copy(ke_h.at[wid], ke_v)
    pltpu.sync_copy(bc_h.at[wid], bc_v)
    bc = bc_v[...][0]

    def fire(j, sem):
        bj = _sgat(bl_v, j)
        pltpu.async_copy(
            tt_hbm.at[:, pl.ds(bj * _CBLK, _CBLK)], ring_v.at[j & 3], sem
        )

    def drain(sem):
        pltpu.make_async_copy(
            tt_hbm.at[:, pl.ds(0, _CBLK)], ring_v.at[0], sem
        ).wait()

    fire(0, sem0)

    @pl.when(1 < bc)
    def _():
        fire(1, sem1)

    def body(j, carry):
        @pl.when(jnp.logical_and(j + 2 < bc, ((j + 2) & 1) == 0))
        def _():
            fire(j + 2, sem0)

        @pl.when(jnp.logical_and(j + 2 < bc, ((j + 2) & 1) == 1))
        def _():
            fire(j + 2, sem1)

        @pl.when((j & 1) == 0)
        def _():
            drain(sem0)

        @pl.when((j & 1) == 1)
        def _():
            drain(sem1)

        ks = _sgat(ks_v, j)
        ke = _sgat(ke_v, j)

        def cbody(k, c2):
            i = _sgat(si_v, k)
            cols = jnp.full((_LANES,), i & (_CBLK - 1), jnp.int32)
            for q in range(_DIM // _LANES):
                rows = lax.iota(jnp.int32, _LANES) + q * _LANES
                v = plsc.load_gather(ring_v.at[j & 3], [rows, cols])
                dst_v[k, pl.ds(q * _LANES, _LANES)] = v
            return c2

        lax.fori_loop(ks, ke, cbody, 0)
        return carry

    lax.fori_loop(0, bc, body, 0)
    pltpu.sync_copy(dst_v, outs_hbm.at[wid])


@jax.jit
def _sc_gather(blist, silist, kst, ken, bcnt, tt):
    mesh = plsc.VectorSubcoreMesh(core_axis_name="c", subcore_axis_name="s")
    call = functools.partial(
        pl.kernel,
        mesh=mesh,
        out_type=jax.ShapeDtypeStruct(
            (_NUM_WORKERS, _ROWS_PER_W, _DIM), jnp.float32
        ),
        scratch_types=[
            pltpu.VMEM((_ROWS_PER_W,), jnp.int32),
            pltpu.VMEM((_ROWS_PER_W,), jnp.int32),
            pltpu.VMEM((_ROWS_PER_W,), jnp.int32),
            pltpu.VMEM((_ROWS_PER_W,), jnp.int32),
            pltpu.VMEM((_LANES,), jnp.int32),
            pltpu.VMEM((_RING, _DIM, _CBLK), jnp.float32),
            pltpu.VMEM((_ROWS_PER_W, _DIM), jnp.float32),
            pltpu.SemaphoreType.DMA,
            pltpu.SemaphoreType.DMA,
        ],
        compiler_params=pltpu.CompilerParams(
            use_tc_tiling_on_sc=True, needs_layout_passes=False
        ),
    )(_gather_body)
    return call(blist, silist, kst, ken, bcnt, tt)


def _unperm_body(inv_hbm, src_hbm, out_hbm, idx_v, rows_v, sem):
    wid = lax.axis_index("s") * _NUM_CORES + lax.axis_index("c")
    pltpu.sync_copy(inv_hbm.at[wid], idx_v)
    copies = [
        pltpu.async_copy(src_hbm.at[idx_v.at[j]], rows_v.at[j], sem)
        for j in range(4)
    ]
    for c in copies:
        c.wait()
    pltpu.sync_copy(rows_v, out_hbm.at[wid])


@jax.jit
def _sc_unpermute(inv, src):
    mesh = plsc.VectorSubcoreMesh(core_axis_name="c", subcore_axis_name="s")
    call = functools.partial(
        pl.kernel,
        mesh=mesh,
        out_type=jax.ShapeDtypeStruct((_NUM_WORKERS, 4, 128, _DIM), jnp.float32),
        scratch_types=[
            pltpu.VMEM((4, 128), jnp.int32),
            pltpu.VMEM((4, 128, _DIM), jnp.float32),
            pltpu.SemaphoreType.DMA,
        ],
        compiler_params=pltpu.CompilerParams(use_tc_tiling_on_sc=False),
    )(_unperm_body)
    return call(inv, src)


def kernel(indices, table):
    idx32 = indices.astype(jnp.int32)
    pos = jnp.arange(_BATCH, dtype=jnp.int32)

    order = jnp.argsort(idx32)
    si = idx32[order]
    b = si >> 7
    pit = pos & (_ROWS_PER_W - 1)
    tile = pos >> 9
    newb = (b != jnp.roll(b, 1)) | (pit == 0)
    slot = jnp.cumsum(newb.reshape(_NUM_WORKERS, _ROWS_PER_W), axis=1).astype(
        jnp.int32
    ).reshape(-1) - 1
    bcount = slot.reshape(_NUM_WORKERS, _ROWS_PER_W)[:, -1:] + 1

    col = jnp.where(newb, slot, _ROWS_PER_W)
    blist = jnp.zeros((_NUM_WORKERS, _ROWS_PER_W), jnp.int32)
    blist = blist.at[tile, col].set(b, mode="drop")
    kst = jnp.full((_NUM_WORKERS, _ROWS_PER_W), _ROWS_PER_W, jnp.int32)
    kst = kst.at[tile, col].set(pit, mode="drop")
    ken = jnp.concatenate(
        [kst[:, 1:], jnp.full((_NUM_WORKERS, 1), _ROWS_PER_W, jnp.int32)],
        axis=1,
    )
    bcnt = jnp.pad(bcount, ((0, 0), (0, _LANES - 1)))
    silist = si.reshape(_NUM_WORKERS, _ROWS_PER_W)

    out_sorted = _sc_gather(blist, silist, kst, ken, bcnt, table.T)

    inv = jnp.zeros((_BATCH,), jnp.int32).at[order].set(pos)
    final = _sc_unpermute(
        inv.reshape(_NUM_WORKERS, 4, 128),
        out_sorted.reshape(_BATCH, _DIM),
    )
    return final.reshape(_BATCH, _DIM)


# speedup vs baseline: 1.0053x; 1.0053x over previous
import functools

import jax
import jax.numpy as jnp
from jax import lax
from jax.experimental import pallas as pl
from jax.experimental.pallas import tpu as pltpu
from jax.experimental.pallas import tpu_sc as plsc

_NUM_CORES = 2
_NUM_SUBCORES = 16
_NUM_WORKERS = _NUM_CORES * _NUM_SUBCORES

_BATCH = 16384
_DIM = 64
_CBLK = 128
_ROWS_PER_W = _BATCH // _NUM_WORKERS
_LANES = 16
_RING = 4


def _sgat(ref, k):
    return plsc.load_gather(ref, [jnp.full((_LANES,), k, jnp.int32)])[0]


def _gather_body(
    bl_h, si_h, ks_h, ke_h, bc_h, tt_hbm, outs_hbm,
    bl_v, si_v, ks_v, ke_v, bc_v, ring_v, dst_v, sem0, sem1,
):
    wid = lax.axis_index("s") * _NUM_CORES + lax.axis_index("c")
    pltpu.sync_copy(bl_h.at[wid], bl_v)
    pltpu.sync_copy(si_h.at[wid], si_v)
    pltpu.sync_copy(ks_h.at[wid], ks_v)
    pltpu.sync_copy(ke_h.at[wid], ke_v)
    pltpu.sync_copy(bc_h.at[wid], bc_v)
    bc = bc_v[...][0]

    def fire(j, sem):
        bj = _sgat(bl_v, j)
        pltpu.async_copy(
            tt_hbm.at[:, pl.ds(bj * _CBLK, _CBLK)], ring_v.at[j & 3], sem
        )

    def drain(sem):
        pltpu.make_async_copy(
            tt_hbm.at[:, pl.ds(0, _CBLK)], ring_v.at[0], sem
        ).wait()

    fire(0, sem0)

    @pl.when(1 < bc)
    def _():
        fire(1, sem1)

    def body(j, carry):
        @pl.when(jnp.logical_and(j + 2 < bc, ((j + 2) & 1) == 0))
        def _():
            fire(j + 2, sem0)

        @pl.when(jnp.logical_and(j + 2 < bc, ((j + 2) & 1) == 1))
        def _():
            fire(j + 2, sem1)

        @pl.when((j & 1) == 0)
        def _():
            drain(sem0)

        @pl.when((j & 1) == 1)
        def _():
            drain(sem1)

        ks = _sgat(ks_v, j)
        ke = _sgat(ke_v, j)

        def cbody(k, c2):
            i = _sgat(si_v, k)
            cols = jnp.full((_LANES,), i & (_CBLK - 1), jnp.int32)
            for q in range(_DIM // _LANES):
                rows = lax.iota(jnp.int32, _LANES) + q * _LANES
                v = plsc.load_gather(ring_v.at[j & 3], [rows, cols])
                dst_v[k, pl.ds(q * _LANES, _LANES)] = v
            return c2

        lax.fori_loop(ks, ke, cbody, 0)
        return carry

    lax.fori_loop(0, bc, body, 0)
    pltpu.sync_copy(dst_v, outs_hbm.at[wid])


@jax.jit
def _sc_gather(blist, silist, kst, ken, bcnt, tt):
    mesh = plsc.VectorSubcoreMesh(core_axis_name="c", subcore_axis_name="s")
    call = functools.partial(
        pl.kernel,
        mesh=mesh,
        out_type=jax.ShapeDtypeStruct(
            (_NUM_WORKERS, _ROWS_PER_W, _DIM), jnp.float32
        ),
        scratch_types=[
            pltpu.VMEM((_ROWS_PER_W,), jnp.int32),
            pltpu.VMEM((_ROWS_PER_W,), jnp.int32),
            pltpu.VMEM((_ROWS_PER_W,), jnp.int32),
            pltpu.VMEM((_ROWS_PER_W,), jnp.int32),
            pltpu.VMEM((_LANES,), jnp.int32),
            pltpu.VMEM((_RING, _DIM, _CBLK), jnp.float32),
            pltpu.VMEM((_ROWS_PER_W, _DIM), jnp.float32),
            pltpu.SemaphoreType.DMA,
            pltpu.SemaphoreType.DMA,
        ],
        compiler_params=pltpu.CompilerParams(
            use_tc_tiling_on_sc=True, needs_layout_passes=False
        ),
    )(_gather_body)
    return call(blist, silist, kst, ken, bcnt, tt)


def _unperm_body(inv_hbm, src_hbm, out_hbm, idx_v, rows_v, sem):
    wid = lax.axis_index("s") * _NUM_CORES + lax.axis_index("c")
    pltpu.sync_copy(inv_hbm.at[wid], idx_v)
    copies = [
        pltpu.async_copy(src_hbm.at[idx_v.at[j]], rows_v.at[j], sem)
        for j in range(4)
    ]
    for c in copies:
        c.wait()
    pltpu.sync_copy(rows_v, out_hbm.at[wid])


@jax.jit
def _sc_unpermute(inv, src):
    mesh = plsc.VectorSubcoreMesh(core_axis_name="c", subcore_axis_name="s")
    call = functools.partial(
        pl.kernel,
        mesh=mesh,
        out_type=jax.ShapeDtypeStruct((_NUM_WORKERS, 4, 128, _DIM), jnp.float32),
        scratch_types=[
            pltpu.VMEM((4, 128), jnp.int32),
            pltpu.VMEM((4, 128, _DIM), jnp.float32),
            pltpu.SemaphoreType.DMA,
        ],
        compiler_params=pltpu.CompilerParams(use_tc_tiling_on_sc=False),
    )(_unperm_body)
    return call(inv, src)


def kernel(indices, table):
    idx32 = indices.astype(jnp.int32)
    pos = jnp.arange(_BATCH, dtype=jnp.int32)

    packed = jnp.sort((idx32 >> 7) * _BATCH + pos)
    order = packed & (_BATCH - 1)
    si = idx32[order]
    b = packed >> 14
    pit = pos & (_ROWS_PER_W - 1)
    tile = pos >> 9
    newb = (b != jnp.roll(b, 1)) | (pit == 0)
    slot = jnp.cumsum(newb.reshape(_NUM_WORKERS, _ROWS_PER_W), axis=1).astype(
        jnp.int32
    ).reshape(-1) - 1
    bcount = slot.reshape(_NUM_WORKERS, _ROWS_PER_W)[:, -1:] + 1

    col = jnp.where(newb, slot, _ROWS_PER_W)
    blist = jnp.zeros((_NUM_WORKERS, _ROWS_PER_W), jnp.int32)
    blist = blist.at[tile, col].set(b, mode="drop")
    kst = jnp.full((_NUM_WORKERS, _ROWS_PER_W), _ROWS_PER_W, jnp.int32)
    kst = kst.at[tile, col].set(pit, mode="drop")
    ken = jnp.concatenate(
        [kst[:, 1:], jnp.full((_NUM_WORKERS, 1), _ROWS_PER_W, jnp.int32)],
        axis=1,
    )
    bcnt = jnp.pad(bcount, ((0, 0), (0, _LANES - 1)))
    silist = si.reshape(_NUM_WORKERS, _ROWS_PER_W)

    out_sorted = _sc_gather(blist, silist, kst, ken, bcnt, table.T)

    inv = jnp.zeros((_BATCH,), jnp.int32).at[order].set(pos)
    final = _sc_unpermute(
        inv.reshape(_NUM_WORKERS, 4, 128),
        out_sorted.reshape(_BATCH, _DIM),
    )
    return final.reshape(_BATCH, _DIM)


# speedup vs baseline: 1.5584x; 1.5502x over previous
import functools

import jax
import jax.numpy as jnp
from jax import lax
from jax.experimental import pallas as pl
from jax.experimental.pallas import tpu as pltpu
from jax.experimental.pallas import tpu_sc as plsc

_NUM_CORES = 2
_NUM_SUBCORES = 16
_NUM_WORKERS = _NUM_CORES * _NUM_SUBCORES

_BATCH = 16384
_DIM = 64
_CBLK = 128
_ROWS_PER_W = _BATCH // _NUM_WORKERS
_LANES = 16
_RING = 4


def _sgat(ref, k):
    return plsc.load_gather(ref, [jnp.full((_LANES,), k, jnp.int32)])[0]


def _gather_body(
    bl_h, si_h, ks_h, ke_h, bc_h, tt_hbm, outs_hbm,
    bl_v, si_v, ks_v, ke_v, bc_v, ring_v, dst_v, sem0, sem1,
):
    wid = lax.axis_index("s") * _NUM_CORES + lax.axis_index("c")
    pltpu.sync_copy(bl_h.at[wid], bl_v)
    pltpu.sync_copy(si_h.at[wid], si_v)
    pltpu.sync_copy(ks_h.at[wid], ks_v)
    pltpu.sync_copy(ke_h.at[wid], ke_v)
    pltpu.sync_copy(bc_h.at[wid], bc_v)
    bc = bc_v[...][0]

    def fire(j, sem):
        bj = _sgat(bl_v, j)
        pltpu.async_copy(
            tt_hbm.at[:, pl.ds(bj * _CBLK, _CBLK)], ring_v.at[j & 3], sem
        )

    def drain(sem):
        pltpu.make_async_copy(
            tt_hbm.at[:, pl.ds(0, _CBLK)], ring_v.at[0], sem
        ).wait()

    fire(0, sem0)

    @pl.when(1 < bc)
    def _():
        fire(1, sem1)

    def body(j, carry):
        @pl.when(jnp.logical_and(j + 2 < bc, ((j + 2) & 1) == 0))
        def _():
            fire(j + 2, sem0)

        @pl.when(jnp.logical_and(j + 2 < bc, ((j + 2) & 1) == 1))
        def _():
            fire(j + 2, sem1)

        @pl.when((j & 1) == 0)
        def _():
            drain(sem0)

        @pl.when((j & 1) == 1)
        def _():
            drain(sem1)

        ks = _sgat(ks_v, j)
        ke = _sgat(ke_v, j)

        def cbody(k, c2):
            i = _sgat(si_v, k)
            cols = jnp.full((_LANES,), i & (_CBLK - 1), jnp.int32)
            for q in range(_DIM // _LANES):
                rows = lax.iota(jnp.int32, _LANES) + q * _LANES
                v = plsc.load_gather(ring_v.at[j & 3], [rows, cols])
                dst_v[k, pl.ds(q * _LANES, _LANES)] = v
            return c2

        lax.fori_loop(ks, ke, cbody, 0)
        return carry

    lax.fori_loop(0, bc, body, 0)
    pltpu.sync_copy(dst_v, outs_hbm.at[wid])


@jax.jit
def _sc_gather(blist, silist, kst, ken, bcnt, tt):
    mesh = plsc.VectorSubcoreMesh(core_axis_name="c", subcore_axis_name="s")
    call = functools.partial(
        pl.kernel,
        mesh=mesh,
        out_type=jax.ShapeDtypeStruct(
            (_NUM_WORKERS, _ROWS_PER_W, _DIM), jnp.float32
        ),
        scratch_types=[
            pltpu.VMEM((_ROWS_PER_W,), jnp.int32),
            pltpu.VMEM((_ROWS_PER_W,), jnp.int32),
            pltpu.VMEM((_ROWS_PER_W,), jnp.int32),
            pltpu.VMEM((_ROWS_PER_W,), jnp.int32),
            pltpu.VMEM((_LANES,), jnp.int32),
            pltpu.VMEM((_RING, _DIM, _CBLK), jnp.float32),
            pltpu.VMEM((_ROWS_PER_W, _DIM), jnp.float32),
            pltpu.SemaphoreType.DMA,
            pltpu.SemaphoreType.DMA,
        ],
        compiler_params=pltpu.CompilerParams(
            use_tc_tiling_on_sc=True, needs_layout_passes=False
        ),
    )(_gather_body)
    return call(blist, silist, kst, ken, bcnt, tt)


def _unperm_body(ordr_hbm, src_hbm, out_hbm, idx_v, rows_v, sem):
    wid = lax.axis_index("s") * _NUM_CORES + lax.axis_index("c")
    pltpu.sync_copy(ordr_hbm.at[wid], idx_v)
    pltpu.sync_copy(src_hbm.at[wid], rows_v)
    copies = [
        pltpu.async_copy(rows_v.at[j], out_hbm.at[idx_v.at[j]], sem)
        for j in range(4)
    ]
    for c in copies:
        c.wait()


@jax.jit
def _sc_unpermute(ordr, src):
    mesh = plsc.VectorSubcoreMesh(core_axis_name="c", subcore_axis_name="s")
    call = functools.partial(
        pl.kernel,
        mesh=mesh,
        out_type=jax.ShapeDtypeStruct((_BATCH, _DIM), jnp.float32),
        scratch_types=[
            pltpu.VMEM((4, 128), jnp.int32),
            pltpu.VMEM((4, 128, _DIM), jnp.float32),
            pltpu.SemaphoreType.DMA,
        ],
        compiler_params=pltpu.CompilerParams(use_tc_tiling_on_sc=False),
    )(_unperm_body)
    return call(ordr, src)


def kernel(indices, table):
    idx32 = indices.astype(jnp.int32)
    pos = jnp.arange(_BATCH, dtype=jnp.int32)

    packed = jnp.sort((idx32 >> 7) * _BATCH + pos)
    order = packed & (_BATCH - 1)
    si = idx32[order]
    b = packed >> 14
    pit = pos & (_ROWS_PER_W - 1)
    tile = pos >> 9
    newb = (b != jnp.roll(b, 1)) | (pit == 0)
    slot2 = jnp.cumsum(
        newb.reshape(_NUM_WORKERS, _ROWS_PER_W), axis=1
    ).astype(jnp.int32) - 1
    bcount = slot2[:, -1:] + 1

    jj = jnp.arange(_ROWS_PER_W, dtype=jnp.int32)
    kst = jnp.sum(slot2[:, :, None] < jj[None, None, :], axis=1, dtype=jnp.int32)
    ken = jnp.sum(slot2[:, :, None] <= jj[None, None, :], axis=1, dtype=jnp.int32)
    b2 = b.reshape(_NUM_WORKERS, _ROWS_PER_W)
    blist = jnp.take_along_axis(b2, jnp.clip(kst, 0, _ROWS_PER_W - 1), axis=1)
    bcnt = jnp.pad(bcount, ((0, 0), (0, _LANES - 1)))
    silist = si.reshape(_NUM_WORKERS, _ROWS_PER_W)

    out_sorted = _sc_gather(blist, silist, kst, ken, bcnt, table.T)

    final = _sc_unpermute(
        order.reshape(_NUM_WORKERS, 4, 128),
        out_sorted.reshape(_NUM_WORKERS, 4, 128, _DIM),
    )
    return final
